# baseline (speedup 1.0000x reference)
import functools

import jax
import jax.numpy as jnp
from jax.experimental import pallas as pl

PI = 3.14159265358979


def _acos(x):
    t = jnp.abs(x)
    u = jnp.maximum(1.0 - t, 1e-20)
    s = u * jax.lax.rsqrt(u)
    p = s * (1.56977681 + t * (-0.20193291 + t * 0.04852035))
    return jnp.where(x < 0, PI - p, p)


def _tile_kernel(fi_ref, fj_ref, d_ref, s_ref, a_ref):
    fi = fi_ref[0]
    fj = fj_ref[0]

    pxi, pyi, pzi = fi[:, 0:1], fi[:, 1:2], fi[:, 2:3]
    nxi, nyi, nzi = fi[:, 3:4], fi[:, 4:5], fi[:, 5:6]
    pxj, pyj, pzj = fj[0:1, :], fj[1:2, :], fj[2:3, :]
    nxj, nyj, nzj = fj[3:4, :], fj[4:5, :], fj[5:6, :]

    dx = pxj - pxi
    dy = pyj - pyi
    dz = pzj - pzi
    ln25 = dx * dx + dy * dy + dz * dz
    rln = jax.lax.rsqrt(ln25 + 1e-10)
    d_ref[0] = (ln25 * rln).astype(d_ref.dtype)

    dot1 = nxi * dx + nyi * dy + nzi * dz
    dot2 = nxj * dx + nyj * dy + nzj * dz
    c1 = dot1 * rln
    c2 = dot2 * rln
    a_ref[0] = jnp.abs(_acos(c1) + _acos(c2) - PI).astype(a_ref.dtype)

    cs = nxi * nxj + nyi * nyj + nzi * nzj
    cs = jnp.where(jnp.isnan(cs), 0.0, cs)
    s_ref[0] = _acos(cs).astype(s_ref.dtype)


@functools.partial(jax.jit, static_argnames=("interpret",))
def _run(points, normals, interpret=False):
    B, N, _ = points.shape
    nh = normals * jax.lax.rsqrt(
        jnp.sum(normals * normals, axis=-1, keepdims=True))
    zeros = jnp.zeros((B, N, 2), points.dtype)
    feat_i = jnp.concatenate([points * 5.0, nh, zeros], axis=-1)
    feat_j = jnp.swapaxes(feat_i, 1, 2)

    RI = 128
    plane = jax.ShapeDtypeStruct((B, N, N), jnp.bfloat16)
    d, s, a = pl.pallas_call(
        _tile_kernel,
        grid=(B, N // RI),
        in_specs=[
            pl.BlockSpec((1, RI, 8), lambda b, r: (b, r, 0)),
            pl.BlockSpec((1, 8, N), lambda b, r: (b, 0, 0)),
        ],
        out_specs=[
            pl.BlockSpec((1, RI, N), lambda b, r: (b, r, 0)),
            pl.BlockSpec((1, RI, N), lambda b, r: (b, r, 0)),
            pl.BlockSpec((1, RI, N), lambda b, r: (b, r, 0)),
        ],
        out_shape=[plane, plane, plane],
        interpret=interpret,
    )(feat_i, feat_j)
    return jnp.stack([d, s, a], axis=-1).astype(points.dtype)


def kernel(points, normals, add_num):
    return _run(points, normals)


# speedup vs baseline: 1.0067x; 1.0067x over previous
import functools

import jax
import jax.numpy as jnp
from jax.experimental import pallas as pl

PI = 3.14159265358979


def _acos(x):
    t = jnp.abs(x)
    u = jnp.maximum(1.0 - t, 1e-20)
    s = u * jax.lax.rsqrt(u)
    p = s * (1.56977681 + t * (-0.20193291 + t * 0.04852035))
    return jnp.where(x < 0, PI - p, p)


def _tile_kernel(fi_ref, fj_ref, d_ref, s_ref, a_ref):
    fi = fi_ref[0]
    fj = fj_ref[0]

    pxi, pyi, pzi = fi[:, 0:1], fi[:, 1:2], fi[:, 2:3]
    nxi, nyi, nzi = fi[:, 3:4], fi[:, 4:5], fi[:, 5:6]
    pxj, pyj, pzj = fj[0:1, :], fj[1:2, :], fj[2:3, :]
    nxj, nyj, nzj = fj[3:4, :], fj[4:5, :], fj[5:6, :]

    dx = pxj - pxi
    dy = pyj - pyi
    dz = pzj - pzi
    ln25 = dx * dx + dy * dy + dz * dz
    rln = jax.lax.rsqrt(ln25 + 1e-10)
    d_ref[0] = (ln25 * rln).astype(d_ref.dtype)

    dot1 = nxi * dx + nyi * dy + nzi * dz
    dot2 = nxj * dx + nyj * dy + nzj * dz
    c1 = dot1 * rln
    c2 = dot2 * rln
    a_ref[0] = jnp.abs(_acos(c1) + _acos(c2) - PI).astype(a_ref.dtype)

    cs = nxi * nxj + nyi * nyj + nzi * nzj
    cs = jnp.where(jnp.isnan(cs), 0.0, cs)
    s_ref[0] = _acos(cs).astype(s_ref.dtype)


@functools.partial(jax.jit, static_argnames=("interpret",))
def _run(points, normals, interpret=False):
    B, N, _ = points.shape
    nh = normals * jax.lax.rsqrt(
        jnp.sum(normals * normals, axis=-1, keepdims=True))
    zeros = jnp.zeros((B, N, 2), points.dtype)
    feat_i = jnp.concatenate([points * 5.0, nh, zeros], axis=-1)
    feat_j = jnp.swapaxes(feat_i, 1, 2)

    RI = 256
    plane = jax.ShapeDtypeStruct((B, N, N), jnp.bfloat16)
    d, s, a = pl.pallas_call(
        _tile_kernel,
        grid=(B, N // RI),
        in_specs=[
            pl.BlockSpec((1, RI, 8), lambda b, r: (b, r, 0)),
            pl.BlockSpec((1, 8, N), lambda b, r: (b, 0, 0)),
        ],
        out_specs=[
            pl.BlockSpec((1, RI, N), lambda b, r: (b, r, 0)),
            pl.BlockSpec((1, RI, N), lambda b, r: (b, r, 0)),
            pl.BlockSpec((1, RI, N), lambda b, r: (b, r, 0)),
        ],
        out_shape=[plane, plane, plane],
        interpret=interpret,
    )(feat_i, feat_j)
    return jnp.stack([d, s, a], axis=-1).astype(points.dtype)


def kernel(points, normals, add_num):
    return _run(points, normals)
